# baseline (speedup 1.0000x reference)
import functools

import jax
import jax.numpy as jnp
from jax import lax
from jax.experimental import pallas as pl
from jax.experimental.pallas import tpu as pltpu
from jax.experimental.pallas import tpu_sc as plsc

NC = 2
NS = 16
NW = NC * NS
L = 16

N_PAD = 10240
R = N_PAD // NW
D = 256
CH1 = 2048
CH2 = 1024
G = 64
SENT = 1 << 20


def _matmul_body(x_ref, wt_ref, o_ref):
    o_ref[...] = jnp.dot(x_ref[...], wt_ref[...],
                         preferred_element_type=jnp.float32)


def _matmul(x, Wt):
    M, K = x.shape
    N = Wt.shape[1]
    BM = 1000
    return pl.pallas_call(
        _matmul_body,
        grid=(M // BM,),
        in_specs=[
            pl.BlockSpec((BM, K), lambda i: (i, 0)),
            pl.BlockSpec((K, N), lambda i: (0, 0)),
        ],
        out_specs=pl.BlockSpec((BM, N), lambda i: (i, 0)),
        out_shape=jax.ShapeDtypeStruct((M, N), jnp.float32),
    )(x, Wt)


def _newton_rsqrt(x):
    s = 0.5 * (x + 1.0)
    for _ in range(14):
        s = 0.5 * (s + x / s)
    return 1.0 / s


def _make_deg_kernel(e1p):
    mesh = plsc.VectorSubcoreMesh(core_axis_name="c", subcore_axis_name="s")

    @functools.partial(
        pl.kernel,
        out_type=jax.ShapeDtypeStruct((N_PAD,), jnp.float32),
        mesh=mesh,
        compiler_params=pltpu.CompilerParams(needs_layout_passes=False),
        scratch_types=[
            pltpu.VMEM((CH1,), jnp.int32),
            pltpu.VMEM((L * R,), jnp.float32),
            pltpu.VMEM((R,), jnp.float32),
        ],
    )
    def deg_kernel(dst_hbm, dis_hbm, dbuf, acc, dis_v):
        wid = lax.axis_index("s") * NC + lax.axis_index("c")
        base = wid * R
        lane = lax.iota(jnp.int32, L)
        lane_off = lane * R
        ones = jnp.ones((L,), jnp.float32)

        def zero_body(i, _):
            acc[pl.ds(i * L, L)] = jnp.zeros((L,), jnp.float32)
            return 0

        lax.fori_loop(0, (L * R) // L, zero_body, 0)

        def chunk_body(c, _):
            pltpu.sync_copy(dst_hbm.at[pl.ds(c * CH1, CH1)], dbuf)

            def step(j, _):
                vd = dbuf[pl.ds(j * L, L)]
                m = (vd >= base) & (vd < base + R)
                idx = jnp.where(m, vd - base, 0) + lane_off
                plsc.addupdate_scatter(acc, [idx], ones, mask=m)
                return 0

            lax.fori_loop(0, CH1 // L, step, 0)
            return 0

        lax.fori_loop(0, e1p // CH1, chunk_body, 0)

        def red_body(r, _):
            s = acc[pl.ds(r * L, L)]
            for l in range(1, L):
                s = s + acc[pl.ds(l * R + r * L, L)]
            deg = s + 1.0
            dis_v[pl.ds(r * L, L)] = _newton_rsqrt(deg)
            return 0

        lax.fori_loop(0, R // L, red_body, 0)
        pltpu.sync_copy(dis_v, dis_hbm.at[pl.ds(base, R)])

    return deg_kernel


def _make_agg_kernel(e2p):
    mesh = plsc.VectorSubcoreMesh(core_axis_name="c", subcore_axis_name="s")

    @functools.partial(
        pl.kernel,
        out_type=jax.ShapeDtypeStruct((N_PAD, D), jnp.float32),
        mesh=mesh,
        compiler_params=pltpu.CompilerParams(needs_layout_passes=False),
        scratch_types=[
            pltpu.VMEM((CH2,), jnp.int32),
            pltpu.VMEM((CH2,), jnp.int32),
            pltpu.VMEM((CH2 + G,), jnp.int32),
            pltpu.VMEM((CH2 + G,), jnp.int32),
            pltpu.VMEM((CH2 + G,), jnp.float32),
            pltpu.VMEM((N_PAD,), jnp.float32),
            pltpu.VMEM((R + 1, D), jnp.float32),
            pltpu.VMEM((G, D), jnp.float32),
            pltpu.VMEM((D,), jnp.float32),
            pltpu.SemaphoreType.DMA,
        ],
    )
    def agg_kernel(src_hbm, dst_hbm, h_hbm, dis_hbm, b_hbm, out_hbm,
                   sbuf, dbuf, wl_src, wl_loc, wl_w, dis_v, acc, rows,
                   b_v, sem):
        wid = lax.axis_index("s") * NC + lax.axis_index("c")
        base = wid * R
        lane = lax.iota(jnp.int32, L)
        zeros_l = jnp.zeros((L,), jnp.float32)
        zeros_i = jnp.zeros((L,), jnp.int32)

        pltpu.sync_copy(dis_hbm, dis_v)
        pltpu.sync_copy(b_hbm, b_v)

        def zero_body(i, _):
            for jj in range(D // L):
                acc[i, pl.ds(jj * L, L)] = zeros_l
            return 0

        lax.fori_loop(0, R + 1, zero_body, 0)

        def chunk_body(c, _):
            pltpu.sync_copy(src_hbm.at[pl.ds(c * CH2, CH2)], sbuf)
            pltpu.sync_copy(dst_hbm.at[pl.ds(c * CH2, CH2)], dbuf)

            def scan_step(j, cnt):
                vs = sbuf[pl.ds(j * L, L)]
                vd = dbuf[pl.ds(j * L, L)]
                m = (vd >= base) & (vd < base + R)
                mi = m.astype(jnp.int32)
                pos = cnt + plsc.cumsum(mi) - mi
                plsc.store_scatter(wl_src, [pos], vs, mask=m)
                plsc.store_scatter(wl_loc, [pos], vd - base, mask=m)
                wv = plsc.load_gather(dis_v, [vs])
                plsc.store_scatter(wl_w, [pos], wv, mask=m)
                return cnt + lax.reduce_sum(mi, axes=(0,))

            cnt = lax.fori_loop(0, CH2 // L, scan_step, jnp.int32(0))

            nblk = (cnt + G - 1) // G
            for k in range(G // L):
                pidx = cnt + k * L + lane
                pm = pidx < nblk * G
                plsc.store_scatter(wl_src, [pidx], zeros_i, mask=pm)
                plsc.store_scatter(wl_loc, [pidx],
                                   jnp.full((L,), R, jnp.int32), mask=pm)
                plsc.store_scatter(wl_w, [pidx], zeros_l, mask=pm)

            def gather_body(g, _):
                pltpu.async_copy(
                    h_hbm.at[wl_src.at[pl.ds(g * G, G)]], rows, sem
                ).wait()

                def row_body(r, _):
                    quad = r & ~(L - 1)
                    lm = lane == (r - quad)
                    wv16 = wl_w[pl.ds(g * G + quad, L)]
                    lv16 = wl_loc[pl.ds(g * G + quad, L)]
                    w = lax.reduce_sum(jnp.where(lm, wv16, 0.0), axes=(0,))
                    loc = lax.reduce_sum(jnp.where(lm, lv16, 0), axes=(0,))
                    for jj in range(D // L):
                        sl = pl.ds(jj * L, L)
                        acc[loc, sl] = acc[loc, sl] + w * rows[r, sl]
                    return 0

                lax.fori_loop(0, G, row_body, 0)
                return 0

            lax.fori_loop(0, nblk, gather_body, 0)
            return 0

        lax.fori_loop(0, e2p // CH2, chunk_body, 0)

        def out_blk(blk, _):
            def out_row(r, _):
                quad = r & ~(L - 1)
                lm = lane == (r - quad)
                dv16 = dis_v[pl.ds(base + blk * G + quad, L)]
                d = lax.reduce_sum(jnp.where(lm, dv16, 0.0), axes=(0,))
                i = blk * G + r
                for jj in range(D // L):
                    sl = pl.ds(jj * L, L)
                    rows[r, sl] = jnp.maximum(d * acc[i, sl] + b_v[sl], 0.0)
                return 0

            lax.fori_loop(0, G, out_row, 0)
            pltpu.sync_copy(rows, out_hbm.at[pl.ds(base + blk * G, G)])
            return 0

        lax.fori_loop(0, R // G, out_blk, 0)

    return agg_kernel


def _pad_to(a, n, fill):
    return jnp.concatenate(
        [a, jnp.full((n - a.shape[0],), fill, dtype=a.dtype)]
    )


def kernel(x, edge_index, W, b):
    N = x.shape[0]
    edge_index = edge_index.astype(jnp.int32)
    src = edge_index[0]
    dst = edge_index[1]
    e1 = dst.shape[0]
    e1p = pl.cdiv(e1, CH1) * CH1
    dst1p = _pad_to(dst, e1p, SENT)

    loop = jnp.arange(N, dtype=jnp.int32)
    src2 = jnp.concatenate([src, loop])
    dst2 = jnp.concatenate([dst, loop])
    e2 = src2.shape[0]
    e2p = pl.cdiv(e2, CH2) * CH2
    src2p = _pad_to(src2, e2p, 0)
    dst2p = _pad_to(dst2, e2p, SENT)

    h = _matmul(x, W.T)
    dis = _make_deg_kernel(e1p)(dst1p)
    out = _make_agg_kernel(e2p)(src2p, dst2p, h, dis, b)
    return out[:N]
